# Initial kernel scaffold; baseline (speedup 1.0000x reference)
import jax
import jax.numpy as jnp
from jax.experimental import pallas as pl


def kernel(input_ids, weight, hash_a, hash_b, sign_a, sign_b):
    raise NotImplementedError("write your pallas kernel here")


# speedup vs baseline: 4.3158x; 4.3158x over previous
import functools

import jax
import jax.numpy as jnp
from jax import lax
from jax.experimental import pallas as pl
from jax.experimental.pallas import tpu as pltpu
from jax.experimental.pallas import tpu_sc as plsc

BUCKET_N = 1000000
DIM = 32
NUM_CORES = 2
NUM_SUBCORES = 16
NUM_WORKERS = NUM_CORES * NUM_SUBCORES
LANES = 16
CHUNK = 512
SUB = 128


@functools.lru_cache(maxsize=None)
def _make_launcher(n_ids):
    per_w = n_ids // NUM_WORKERS
    n_chunks = per_w // CHUNK
    assert n_chunks % 2 == 0
    mesh = plsc.VectorSubcoreMesh(core_axis_name="c", subcore_axis_name="s")

    buf_shapes = [
        pltpu.VMEM((CHUNK,), jnp.int32),
        pltpu.VMEM((CHUNK // SUB, SUB), jnp.int32),
        pltpu.VMEM((CHUNK // SUB, SUB), jnp.int32),
        pltpu.VMEM((CHUNK,), jnp.float32),
        pltpu.VMEM((CHUNK,), jnp.float32),
        pltpu.VMEM((CHUNK, DIM), jnp.float32),
        pltpu.VMEM((CHUNK, DIM), jnp.float32),
        pltpu.VMEM((CHUNK, DIM), jnp.float32),
        pltpu.SemaphoreType.DMA,
        pltpu.SemaphoreType.DMA,
    ]

    @functools.partial(
        pl.kernel,
        mesh=mesh,
        out_type=jax.ShapeDtypeStruct((n_ids, DIM), jnp.float32),
        compiler_params=pltpu.CompilerParams(
            needs_layout_passes=False, use_tc_tiling_on_sc=False),
        scratch_types=[pltpu.VMEM((LANES, LANES), jnp.int32)] + buf_shapes * 2,
    )
    def launch(ids_hbm, w_hbm, par_hbm, out_hbm, par_v, *bufs):
        ba = bufs[:10]
        bb = bufs[10:]
        wid = lax.axis_index("s") * NUM_CORES + lax.axis_index("c")
        base = wid * per_w
        pltpu.sync_copy(par_hbm, par_v)
        a0 = par_v[0, :]
        b0 = par_v[1, :]
        a1 = par_v[2, :]
        b1 = par_v[3, :]
        sa0 = par_v[4, :]
        sb0 = par_v[5, :]
        sa1 = par_v[6, :]
        sb1 = par_v[7, :]

        def floor_mod(v):
            r = lax.rem(v, BUCKET_N)
            return jnp.where(r < 0, r + BUCKET_N, r)

        def prep(c, buf):
            ids_v, idx0_v, idx1_v, s0_v, s1_v, rows0_v, rows1_v, _, gsem, _ = buf
            off = base + c * CHUNK
            pltpu.sync_copy(ids_hbm.at[pl.ds(off, CHUNK)], ids_v)

            def hash_body(i, _):
                idv = ids_v[pl.ds(i * LANES, LANES)]
                r0 = floor_mod(idv * a0 + b0)
                r1 = floor_mod(idv * a1 + b1)
                u0 = idv * sa0 + sb0
                u1 = idv * sa1 + sb1
                f0 = jnp.where((u0 & 1) == 1, 0.5, -0.5).astype(jnp.float32)
                f1 = jnp.where((u1 & 1) == 1, 0.5, -0.5).astype(jnp.float32)
                j = i // (SUB // LANES)
                k = i % (SUB // LANES)
                idx0_v[j, pl.ds(k * LANES, LANES)] = r0
                idx1_v[j, pl.ds(k * LANES, LANES)] = r1
                s0_v[pl.ds(i * LANES, LANES)] = f0
                s1_v[pl.ds(i * LANES, LANES)] = f1
                return 0

            lax.fori_loop(0, CHUNK // LANES, hash_body, 0)
            for j in range(CHUNK // SUB):
                pltpu.async_copy(w_hbm.at[idx0_v.at[j]],
                                 rows0_v.at[pl.ds(j * SUB, SUB)], gsem)
                pltpu.async_copy(w_hbm.at[idx1_v.at[j]],
                                 rows1_v.at[pl.ds(j * SUB, SUB)], gsem)

        def drain_gathers(buf):
            rows0_v, rows1_v, gsem = buf[5], buf[6], buf[8]
            pltpu.make_async_copy(
                out_hbm.at[pl.ds(0, CHUNK)], rows0_v, gsem).wait()
            pltpu.make_async_copy(
                out_hbm.at[pl.ds(0, CHUNK)], rows1_v, gsem).wait()

        def drain_out(buf):
            outb_v, osem = buf[7], buf[9]
            pltpu.make_async_copy(
                outb_v, out_hbm.at[pl.ds(0, CHUNK)], osem).wait()

        def combine_and_send(c, buf):
            s0_v, s1_v, rows0_v, rows1_v, outb_v, osem = (
                buf[3], buf[4], buf[5], buf[6], buf[7], buf[9])

            def body(t, _):
                for u in range(4):
                    n = t * 4 + u
                    nf = jnp.full((LANES,), n, jnp.int32)
                    g0 = plsc.load_gather(s0_v, [nf])
                    g1 = plsc.load_gather(s1_v, [nf])
                    for h in (0, LANES):
                        outb_v[n, pl.ds(h, LANES)] = (
                            rows0_v[n, pl.ds(h, LANES)] * g0
                            + rows1_v[n, pl.ds(h, LANES)] * g1)
                return 0

            lax.fori_loop(0, CHUNK // 4, body, 0)
            off = base + c * CHUNK
            pltpu.async_copy(outb_v, out_hbm.at[pl.ds(off, CHUNK)], osem)

        prep(0, ba)

        def pair_body(q, _):
            c = 2 * q
            prep(c + 1, bb)
            drain_gathers(ba)

            @pl.when(q > 0)
            def _():
                drain_out(ba)

            combine_and_send(c, ba)

            @pl.when(c + 2 < n_chunks)
            def _():
                prep(c + 2, ba)

            drain_gathers(bb)

            @pl.when(q > 0)
            def _():
                drain_out(bb)

            combine_and_send(c + 1, bb)
            return 0

        lax.fori_loop(0, n_chunks // 2, pair_body, 0)
        drain_out(ba)
        drain_out(bb)

    return launch


def kernel(input_ids, weight, hash_a, hash_b, sign_a, sign_b):
    b, t = input_ids.shape
    n = b * t
    ids = input_ids.reshape(n).astype(jnp.int32)
    par = jnp.stack([hash_a[0], hash_b[0], hash_a[1], hash_b[1],
                     sign_a[0], sign_b[0], sign_a[1], sign_b[1]])
    par = jnp.concatenate([par.astype(jnp.int32), jnp.zeros((8,), jnp.int32)])
    par = jnp.broadcast_to(par[:, None], (LANES, LANES))
    out = _make_launcher(n)(ids, weight.astype(jnp.float32), par)
    return out.reshape(b, t, DIM)
